# baseline (speedup 1.0000x reference)
import functools

import jax
import jax.numpy as jnp
from jax import lax
from jax.experimental import pallas as pl
from jax.experimental.pallas import tpu as pltpu
from jax.experimental.pallas import tpu_sc as plsc

EMBED_DIM = 64
NUM_BINS = 50
VOCAB = NUM_BINS + 3
B = 4096
G = 200
N = B * G

NC = 2
NS = 16
NW = NC * NS
PER_W = N // NW
CHUNK = 512
N_CHUNKS = PER_W // CHUNK


def _sc_kernel(idx_hbm, norm_hbm, table_hbm, w_hbm, b_hbm, out_hbm,
               table_v, w_v, b_v, idx_v, norm_v, out_v):
    wid = lax.axis_index("s") * NC + lax.axis_index("c")
    base = wid * PER_W

    pltpu.sync_copy(table_hbm, table_v)
    pltpu.sync_copy(w_hbm, w_v)
    pltpu.sync_copy(b_hbm, b_v)

    def fold_b(i, _):
        for j in range(4):
            s = pl.ds(i * EMBED_DIM + j * 16, 16)
            table_v[s] = table_v[s] + b_v[pl.ds(j * 16, 16)]
        return 0
    lax.fori_loop(0, VOCAB, fold_b, 0)

    ws = tuple(w_v[pl.ds(16 * j, 16)] for j in range(4))
    iota = lax.iota(jnp.int32, 16)
    iotas = tuple(iota + 16 * j for j in range(4))

    def chunk_body(k, _):
        off = base + k * CHUNK
        pltpu.sync_copy(idx_hbm.at[pl.ds(off, CHUNK)], idx_v)
        pltpu.sync_copy(norm_hbm.at[pl.ds(off, CHUNK)], norm_v)

        @plsc.parallel_loop(0, CHUNK // 16, unroll=2)
        def group_body(g):
            iv = idx_v[pl.ds(g * 16, 16)]
            nv = norm_v[pl.ds(g * 16, 16)]
            ivx = iv * EMBED_DIM
            for l in range(16):
                tb = jnp.broadcast_to(ivx[l], (16,))
                norm_c = nv[l]
                obase = (g * 16 + l) * EMBED_DIM
                for j in range(4):
                    row = plsc.load_gather(table_v, [tb + iotas[j]])
                    out_v[pl.ds(obase + j * 16, 16)] = row + norm_c * ws[j]

        pltpu.sync_copy(out_v, out_hbm.at[pl.ds(off * EMBED_DIM,
                                                CHUNK * EMBED_DIM)])
        return 0
    lax.fori_loop(0, N_CHUNKS, chunk_body, 0)


@jax.jit
def _run(idx, norm, table, w, b):
    mesh = plsc.VectorSubcoreMesh(core_axis_name="c", subcore_axis_name="s")
    kern = functools.partial(
        pl.kernel,
        mesh=mesh,
        compiler_params=pltpu.CompilerParams(needs_layout_passes=False),
        out_type=jax.ShapeDtypeStruct((N * EMBED_DIM,), jnp.float32),
        scratch_types=[
            pltpu.VMEM((VOCAB * EMBED_DIM,), jnp.float32),
            pltpu.VMEM((EMBED_DIM,), jnp.float32),
            pltpu.VMEM((EMBED_DIM,), jnp.float32),
            pltpu.VMEM((CHUNK,), jnp.int32),
            pltpu.VMEM((CHUNK,), jnp.float32),
            pltpu.VMEM((CHUNK * EMBED_DIM,), jnp.float32),
        ],
    )(_sc_kernel)
    flat = kern(idx, norm, table, w, b)
    return flat.reshape(B, G, EMBED_DIM)


def kernel(discrete_expression, normalized_expr, bin_table, W, b):
    idx = discrete_expression.reshape(-1).astype(jnp.int32)
    norm = normalized_expr.reshape(-1).astype(jnp.float32)
    table = bin_table.reshape(-1)
    w = W.reshape(-1)
    return _run(idx, norm, table, w, b)


# speedup vs baseline: 1.1016x; 1.1016x over previous
import functools

import jax
import jax.numpy as jnp
from jax import lax
from jax.experimental import pallas as pl
from jax.experimental.pallas import tpu as pltpu
from jax.experimental.pallas import tpu_sc as plsc

EMBED_DIM = 64
NUM_BINS = 50
VOCAB = NUM_BINS + 3
B = 4096
G = 200
N = B * G

NC = 2
NS = 16
NW = NC * NS
PER_W = N // NW
CHUNK = 512
N_CHUNKS = PER_W // CHUNK


def _sc_kernel(idx_hbm, norm_hbm, table_hbm, w_hbm, b_hbm, out_hbm,
               table_v, w_v, b_v, idx_v, norm_v, out_v):
    wid = lax.axis_index("s") * NC + lax.axis_index("c")
    base = wid * PER_W

    pltpu.sync_copy(table_hbm, table_v)
    pltpu.sync_copy(w_hbm, w_v)
    pltpu.sync_copy(b_hbm, b_v)

    def fold_b(i, _):
        for j in range(4):
            s = pl.ds(i * EMBED_DIM + j * 16, 16)
            table_v[s] = table_v[s] + b_v[pl.ds(j * 16, 16)]
        return 0
    lax.fori_loop(0, VOCAB, fold_b, 0)

    ws = tuple(w_v[pl.ds(16 * j, 16)] for j in range(4))
    iota = lax.iota(jnp.int32, 16)
    iotas = tuple(iota + 16 * j for j in range(4))

    def chunk_body(k, _):
        off = base + k * CHUNK
        pltpu.sync_copy(idx_hbm.at[pl.ds(off, CHUNK)], idx_v)
        pltpu.sync_copy(norm_hbm.at[pl.ds(off, CHUNK)], norm_v)

        @plsc.parallel_loop(0, CHUNK // 16)
        def group_body(g):
            iv = idx_v[pl.ds(g * 16, 16)]
            nv = norm_v[pl.ds(g * 16, 16)]
            ivx = iv * EMBED_DIM
            for l in range(16):
                tb = jnp.broadcast_to(ivx[l], (16,))
                norm_c = nv[l]
                obase = (g * 16 + l) * EMBED_DIM
                for j in range(4):
                    row = plsc.load_gather(table_v, [tb + iotas[j]])
                    out_v[pl.ds(obase + j * 16, 16)] = row + norm_c * ws[j]

        pltpu.sync_copy(out_v, out_hbm.at[pl.ds(off * EMBED_DIM,
                                                CHUNK * EMBED_DIM)])
        return 0
    lax.fori_loop(0, N_CHUNKS, chunk_body, 0)


@jax.jit
def _run(idx, norm, table, w, b):
    mesh = plsc.VectorSubcoreMesh(core_axis_name="c", subcore_axis_name="s")
    kern = functools.partial(
        pl.kernel,
        mesh=mesh,
        compiler_params=pltpu.CompilerParams(needs_layout_passes=False),
        out_type=jax.ShapeDtypeStruct((N * EMBED_DIM,), jnp.float32),
        scratch_types=[
            pltpu.VMEM((VOCAB * EMBED_DIM,), jnp.float32),
            pltpu.VMEM((EMBED_DIM,), jnp.float32),
            pltpu.VMEM((EMBED_DIM,), jnp.float32),
            pltpu.VMEM((CHUNK,), jnp.int32),
            pltpu.VMEM((CHUNK,), jnp.float32),
            pltpu.VMEM((CHUNK * EMBED_DIM,), jnp.float32),
        ],
    )(_sc_kernel)
    flat = kern(idx, norm, table, w, b)
    return flat.reshape(B, G, EMBED_DIM)


def kernel(discrete_expression, normalized_expr, bin_table, W, b):
    idx = discrete_expression.reshape(-1).astype(jnp.int32)
    norm = normalized_expr.reshape(-1).astype(jnp.float32)
    table = bin_table.reshape(-1)
    w = W.reshape(-1)
    return _run(idx, norm, table, w, b)


# speedup vs baseline: 1.6548x; 1.5022x over previous
import functools

import jax
import jax.numpy as jnp
from jax import lax
from jax.experimental import pallas as pl
from jax.experimental.pallas import tpu as pltpu
from jax.experimental.pallas import tpu_sc as plsc

EMBED_DIM = 64
NUM_BINS = 50
VOCAB = NUM_BINS + 3
B = 4096
G = 200
N = B * G

NC = 2
NS = 16
NW = NC * NS
PER_W = N // NW
CHUNK = 512
N_CHUNKS = PER_W // CHUNK


def _sc_kernel(idx_hbm, norm_hbm, table_hbm, w_hbm, b_hbm, out_hbm,
               table_v, w_v, b_v, idx_v, norm_v, out_v):
    wid = lax.axis_index("s") * NC + lax.axis_index("c")
    base = wid * PER_W

    pltpu.sync_copy(table_hbm, table_v)
    pltpu.sync_copy(w_hbm, w_v)
    pltpu.sync_copy(b_hbm, b_v)

    def fold_b(i, _):
        for j in range(4):
            s = pl.ds(i * EMBED_DIM + j * 16, 16)
            table_v[s] = table_v[s] + b_v[pl.ds(j * 16, 16)]
        return 0
    lax.fori_loop(0, VOCAB, fold_b, 0)

    ws = tuple(w_v[pl.ds(16 * j, 16)] for j in range(4))
    iota = lax.iota(jnp.int32, 16)
    iotas = tuple(iota + 16 * j for j in range(4))

    def chunk_body(k, _):
        off = base + k * CHUNK
        pltpu.sync_copy(idx_hbm.at[pl.ds(off, CHUNK)], idx_v)
        pltpu.sync_copy(norm_hbm.at[pl.ds(off, CHUNK)], norm_v)

        @plsc.parallel_loop(0, CHUNK // 16)
        def group_body(g):
            iv = idx_v[pl.ds(g * 16, 16)]
            nv = norm_v[pl.ds(g * 16, 16)]
            ivx = iv * EMBED_DIM
            for l in range(16):
                tb = jnp.broadcast_to(ivx[l], (16,))
                norm_c = nv[l]
                t = g * 16 + l
                for j in range(4):
                    row = plsc.load_gather(table_v, [tb + iotas[j]])
                    out_v[t, pl.ds(j * 16, 16)] = row + norm_c * ws[j]

        pltpu.sync_copy(out_v, out_hbm.at[pl.ds(off, CHUNK), :])
        return 0
    lax.fori_loop(0, N_CHUNKS, chunk_body, 0)


@jax.jit
def _run(idx, norm, table, w, b):
    mesh = plsc.VectorSubcoreMesh(core_axis_name="c", subcore_axis_name="s")
    kern = functools.partial(
        pl.kernel,
        mesh=mesh,
        compiler_params=pltpu.CompilerParams(needs_layout_passes=False),
        out_type=jax.ShapeDtypeStruct((N, EMBED_DIM), jnp.float32),
        scratch_types=[
            pltpu.VMEM((VOCAB * EMBED_DIM,), jnp.float32),
            pltpu.VMEM((EMBED_DIM,), jnp.float32),
            pltpu.VMEM((EMBED_DIM,), jnp.float32),
            pltpu.VMEM((CHUNK,), jnp.int32),
            pltpu.VMEM((CHUNK,), jnp.float32),
            pltpu.VMEM((CHUNK, EMBED_DIM), jnp.float32),
        ],
    )(_sc_kernel)
    flat = kern(idx, norm, table, w, b)
    return flat.reshape(B, G, EMBED_DIM)


def kernel(discrete_expression, normalized_expr, bin_table, W, b):
    idx = discrete_expression.reshape(-1).astype(jnp.int32)
    norm = normalized_expr.reshape(-1).astype(jnp.float32)
    table = bin_table.reshape(-1)
    w = W.reshape(-1)
    return _run(idx, norm, table, w, b)
